# baseline (speedup 1.0000x reference)
import functools

import jax
import jax.numpy as jnp
from jax import lax
from jax.experimental import pallas as pl
from jax.experimental.pallas import tpu as pltpu
from jax.experimental.pallas import tpu_sc as plsc

_N = 10000
_NP = 10240
_E = 320000
_NC = 2
_NS = 16
_NW = _NC * _NS
_CHUNK = 128
_CPW0 = 120
_CPW1 = 40
_BPG = 40
_EP = _NS * (_CPW0 + _CPW1) * _CHUNK
_RPS = _NP // _NS


def _make_sc_aggregate(dpad, nbuf):
  mesh = plsc.VectorSubcoreMesh(core_axis_name="c", subcore_axis_name="s")

  @functools.partial(
      pl.kernel,
      mesh=mesh,
      compiler_params=pltpu.CompilerParams(use_tc_tiling_on_sc=False),
      out_type=jax.ShapeDtypeStruct((_NC, _N, dpad), jnp.float32),
      scratch_types=(
          [pltpu.VMEM((_BPG, _CHUNK), jnp.int32),
           pltpu.VMEM((_BPG, _CHUNK), jnp.int32)]
          + [pltpu.VMEM((_CHUNK, dpad), jnp.float32) for _ in range(nbuf)]
          + [pltpu.VMEM_SHARED((_NP, dpad), jnp.float32)]
          + [pltpu.SemaphoreType.DMA for _ in range(2 * nbuf)]
      ),
  )
  def agg(table_hbm, src_hbm, dst_hbm, out_hbm, srcb, dstb, *rest):
    rbufs = rest[:nbuf]
    acc_sh = rest[nbuf]
    sgs = rest[nbuf + 1:2 * nbuf + 1]
    sss = rest[2 * nbuf + 1:]
    c = lax.axis_index("c")
    s = lax.axis_index("s")
    base = jnp.where(c == 0, s * _CPW0, _NS * _CPW0 + s * _CPW1)

    def g_desc(k, b):
      return pltpu.make_async_copy(table_hbm.at[srcb.at[k]], rbufs[b], sgs[b])

    def s_desc(k, b):
      return pltpu.make_async_copy(rbufs[b], acc_sh.at[dstb.at[k]], sss[b])

    def load_idx_start(g):
      a = pltpu.make_async_copy(
          src_hbm.at[pl.ds(base + g * _BPG, _BPG)], srcb, sgs[0])
      bb = pltpu.make_async_copy(
          dst_hbm.at[pl.ds(base + g * _BPG, _BPG)], dstb, sgs[1 % nbuf])
      a.start()
      bb.start()
      return a, bb

    def pipe_block():
      for b in range(nbuf):
        g_desc(b, b).start()

      @pl.loop(0, _BPG // nbuf - 1)
      def _(j):
        k = j * nbuf
        for b in range(nbuf):
          g_desc(k + b, b).wait()
          s_desc(k + b, b).start(add=True)
        for b in range(nbuf):
          s_desc(k + b, b).wait()
          g_desc(k + nbuf + b, b).start()

      tail = _BPG - nbuf
      for b in range(nbuf):
        g_desc(tail + b, b).wait()
        s_desc(tail + b, b).start(add=True)
      for b in range(nbuf):
        s_desc(tail + b, b).wait()

    a, bb = load_idx_start(0)

    @pl.loop(0, _CHUNK)
    def _(i):
      @pl.loop(0, dpad, step=16)
      def _(j):
        rbufs[0][i, pl.ds(j, 16)] = jnp.zeros((16,), jnp.float32)

    for r in range(_RPS // _CHUNK):
      pltpu.sync_copy(rbufs[0],
                      acc_sh.at[pl.ds(s * _RPS + r * _CHUNK, _CHUNK)])
    a.wait()
    bb.wait()
    plsc.subcore_barrier()

    pipe_block()

    @pl.when(c == 0)
    def _():
      for g in range(1, _CPW0 // _BPG):
        a, bb = load_idx_start(g)
        a.wait()
        bb.wait()
        pipe_block()

    plsc.subcore_barrier()

    opw = _N // _NS
    pltpu.sync_copy(acc_sh.at[pl.ds(s * opw, opw)],
                    out_hbm.at[c].at[pl.ds(s * opw, opw)])

  return agg


def _lrelu(x):
  return jnp.where(x >= 0, x, 0.01 * x)


def _bn(x):
  m = jnp.mean(x, axis=0, keepdims=True)
  v = jnp.mean((x - m) * (x - m), axis=0, keepdims=True)
  return (x - m) * lax.rsqrt(v + 1e-5)


def _dot(a, b):
  return jnp.dot(a, b, preferred_element_type=jnp.float32,
                 precision=lax.Precision.HIGHEST)


def _tc_layer1a(aggp, cntp, x, Wl1, bl1, Wr1):
  nb = 5
  rb = _N // nb

  def body(aggp_ref, cntp_ref, x_ref, wl_ref, bl_ref, wr_ref, z_ref, inv_ref):
    cnt = cntp_ref[0][:, 0:1] + cntp_ref[1][:, 0:1]
    inv = 1.0 / jnp.maximum(cnt, 1.0)
    mean = (aggp_ref[0] + aggp_ref[1]) * inv
    z = _dot(mean, wl_ref[...]) + bl_ref[...][None, :] + _dot(x_ref[...], wr_ref[...])
    z_ref[...] = _lrelu(z)
    inv_ref[...] = inv

  return pl.pallas_call(
      body,
      grid=(nb,),
      in_specs=[
          pl.BlockSpec((2, rb, 128), lambda i: (0, i, 0)),
          pl.BlockSpec((2, rb, 16), lambda i: (0, i, 0)),
          pl.BlockSpec((rb, 128), lambda i: (i, 0)),
          pl.BlockSpec((128, 320), lambda i: (0, 0)),
          pl.BlockSpec((320,), lambda i: (0,)),
          pl.BlockSpec((128, 320), lambda i: (0, 0)),
      ],
      out_specs=[
          pl.BlockSpec((rb, 320), lambda i: (i, 0)),
          pl.BlockSpec((rb, 1), lambda i: (i, 0)),
      ],
      out_shape=[
          jax.ShapeDtypeStruct((_N, 320), jnp.float32),
          jax.ShapeDtypeStruct((_N, 1), jnp.float32),
      ],
  )(aggp, cntp, x, Wl1, bl1, Wr1)


def _tc_bn1(z):
  def body(z_ref, y_ref):
    y_ref[...] = _bn(z_ref[...])

  return pl.pallas_call(
      body,
      out_shape=jax.ShapeDtypeStruct((_N, 320), jnp.float32),
  )(z)


def _tc_bn2(aggpa, aggpb, xw, inv, bl):
  def body(aggpa_ref, aggpb_ref, xw_ref, inv_ref, bl_ref, y_ref):
    agg = jnp.concatenate(
        [aggpa_ref[0] + aggpa_ref[1],
         aggpb_ref[0][:, :52] + aggpb_ref[1][:, :52]], axis=1)
    y = agg * inv_ref[...] + bl_ref[...][None, :] + xw_ref[...]
    y_ref[...] = _bn(_lrelu(y))

  return pl.pallas_call(
      body,
      out_shape=jax.ShapeDtypeStruct((_N, 180), jnp.float32),
  )(aggpa, aggpb, xw, inv, bl)


def _tc_bn_mid(aggp, xw, inv, bl, d):
  def body(aggp_ref, xw_ref, inv_ref, bl_ref, y_ref):
    agg = aggp_ref[0][:, :d] + aggp_ref[1][:, :d]
    y = agg * inv_ref[...] + bl_ref[...][None, :] + xw_ref[...]
    y_ref[...] = _bn(_lrelu(y))

  return pl.pallas_call(
      body,
      out_shape=jax.ShapeDtypeStruct((_N, d), jnp.float32),
  )(aggp, xw, inv, bl)


def _tc_proj(y, Wl, Wr, splits, pad_to):
  nb = 5
  rb = _N // nb
  din = y.shape[1]
  dl = Wl.shape[1]
  dr = Wr.shape[1]
  outs = [s[1] - s[0] for s in splits[:-1]] + [pad_to]

  def body(y_ref, wl_ref, wr_ref, *out_refs):
    h = _dot(y_ref[...], wl_ref[...])
    for k, (lo, hi) in enumerate(splits):
      part = h[:, lo:hi]
      if k == len(splits) - 1 and pad_to > hi - lo:
        part = jnp.pad(part, ((0, 0), (0, pad_to - (hi - lo))))
      out_refs[k][...] = part
    out_refs[-1][...] = _dot(y_ref[...], wr_ref[...])

  return pl.pallas_call(
      body,
      grid=(nb,),
      in_specs=[
          pl.BlockSpec((rb, din), lambda i: (i, 0)),
          pl.BlockSpec((din, dl), lambda i: (0, 0)),
          pl.BlockSpec((din, dr), lambda i: (0, 0)),
      ],
      out_specs=[pl.BlockSpec((rb, c), lambda i: (i, 0)) for c in outs]
      + [pl.BlockSpec((rb, dr), lambda i: (i, 0))],
      out_shape=[jax.ShapeDtypeStruct((_N, c), jnp.float32) for c in outs]
      + [jax.ShapeDtypeStruct((_N, dr), jnp.float32)],
  )(y, Wl, Wr)


def _tc_final(y4, fW1, fb1, fW2, fb2, fW3, fb3):
  blen = _N // 16

  def body(y_ref, fw1_ref, fb1_ref, fw2_ref, fb2_ref, fw3_ref, fb3_ref,
           out_ref):
    col = lax.broadcasted_iota(jnp.int32, (16, _N), 1) // blen
    row = lax.broadcasted_iota(jnp.int32, (16, _N), 0)
    sel = (col == row).astype(jnp.float32)
    p = _dot(sel, y_ref[...])
    p = _dot(p, fw1_ref[...]) + fb1_ref[...][None, :]
    p = _dot(p, fw2_ref[...]) + fb2_ref[...][None, :]
    p = _dot(p, fw3_ref[...]) + fb3_ref[...][None, :]
    out_ref[...] = p

  return pl.pallas_call(
      body,
      out_shape=jax.ShapeDtypeStruct((16, 1), jnp.float32),
  )(y4, fW1, fb1, fW2, fb2, fW3, fb3)


_agg16 = _make_sc_aggregate(16, 4)
_agg128 = _make_sc_aggregate(128, 2)
_agg96 = _make_sc_aggregate(96, 4)
_agg64 = _make_sc_aggregate(64, 4)


def kernel(x_in, edge_index, Wl1, bl1, Wr1, Wl2, bl2, Wr2, Wl3, bl3, Wr3,
           Wl4, bl4, Wr4, fW1, fb1, fW2, fb2, fW3, fb3):
  src = jnp.reshape(
      jnp.concatenate([edge_index[0],
                       jnp.zeros((_EP - _E,), jnp.int32)]),
      (_EP // _CHUNK, _CHUNK))
  pad_dst = _N + jnp.arange(_EP - _E, dtype=jnp.int32) % (_NP - _N)
  dst = jnp.reshape(
      jnp.concatenate([edge_index[1], pad_dst]),
      (_EP // _CHUNK, _CHUNK))

  ac = _agg16(jnp.ones((_N, 16), jnp.float32), src, dst)
  a1 = _agg128(x_in, src, dst)
  z1, inv = _tc_layer1a(a1, ac, x_in, Wl1, bl1, Wr1)
  y1 = _tc_bn1(z1)
  h2a, h2b, xw2 = _tc_proj(y1, Wl2, Wr2, [(0, 128), (128, 180)], 64)

  a2a = _agg128(h2a, src, dst)
  a2b = _agg64(h2b, src, dst)
  y2 = _tc_bn2(a2a, a2b, xw2, inv, bl2)
  h3p, xw3 = _tc_proj(y2, Wl3, Wr3, [(0, 90)], 96)

  a3 = _agg96(h3p, src, dst)
  y3 = _tc_bn_mid(a3, xw3, inv, bl3, 90)
  h4p, xw4 = _tc_proj(y3, Wl4, Wr4, [(0, 50)], 64)

  a4 = _agg64(h4p, src, dst)
  y4 = _tc_bn_mid(a4, xw4, inv, bl4, 50)
  return _tc_final(y4, fW1, fb1, fW2, fb2, fW3, fb3)


# speedup vs baseline: 1.0271x; 1.0271x over previous
import functools

import jax
import jax.numpy as jnp
from jax import lax
from jax.experimental import pallas as pl
from jax.experimental.pallas import tpu as pltpu
from jax.experimental.pallas import tpu_sc as plsc

_N = 10000
_NP = 10240
_E = 320000
_NC = 2
_NS = 16
_NW = _NC * _NS
_CHUNK = 128
_CPW0 = 120
_CPW1 = 40
_BPG = 40
_EP = _NS * (_CPW0 + _CPW1) * _CHUNK
_RPS = _NP // _NS


def _make_sc_aggregate(dpad, nbuf):
  mesh = plsc.VectorSubcoreMesh(core_axis_name="c", subcore_axis_name="s")

  @functools.partial(
      pl.kernel,
      mesh=mesh,
      compiler_params=pltpu.CompilerParams(use_tc_tiling_on_sc=False),
      out_type=jax.ShapeDtypeStruct((_NC, _N, dpad), jnp.float32),
      scratch_types=(
          [pltpu.VMEM((_BPG, _CHUNK), jnp.int32),
           pltpu.VMEM((_BPG, _CHUNK), jnp.int32)]
          + [pltpu.VMEM((_CHUNK, dpad), jnp.float32) for _ in range(nbuf)]
          + [pltpu.VMEM_SHARED((_NP, dpad), jnp.float32)]
          + [pltpu.SemaphoreType.DMA for _ in range(2 * nbuf)]
      ),
  )
  def agg(table_hbm, src_hbm, dst_hbm, out_hbm, srcb, dstb, *rest):
    rbufs = rest[:nbuf]
    acc_sh = rest[nbuf]
    sgs = rest[nbuf + 1:2 * nbuf + 1]
    sss = rest[2 * nbuf + 1:]
    c = lax.axis_index("c")
    s = lax.axis_index("s")
    base = jnp.where(c == 0, s * _CPW0, _NS * _CPW0 + s * _CPW1)

    def g_desc(k, b):
      return pltpu.make_async_copy(table_hbm.at[srcb.at[k]], rbufs[b], sgs[b])

    def s_desc(k, b):
      return pltpu.make_async_copy(rbufs[b], acc_sh.at[dstb.at[k]], sss[b])

    def load_idx_start(g):
      a = pltpu.make_async_copy(
          src_hbm.at[pl.ds(base + g * _BPG, _BPG)], srcb, sgs[0])
      bb = pltpu.make_async_copy(
          dst_hbm.at[pl.ds(base + g * _BPG, _BPG)], dstb, sgs[1 % nbuf])
      a.start()
      bb.start()
      return a, bb

    def pipe_block():
      for b in range(nbuf):
        g_desc(b, b).start()

      @pl.loop(0, _BPG // nbuf - 1)
      def _(j):
        k = j * nbuf
        for b in range(nbuf):
          g_desc(k + b, b).wait()
          s_desc(k + b, b).start(add=True)
        for b in range(nbuf):
          s_desc(k + b, b).wait()
          g_desc(k + nbuf + b, b).start()

      tail = _BPG - nbuf
      for b in range(nbuf):
        g_desc(tail + b, b).wait()
        s_desc(tail + b, b).start(add=True)
      for b in range(nbuf):
        s_desc(tail + b, b).wait()

    a, bb = load_idx_start(0)

    @pl.loop(0, _CHUNK)
    def _(i):
      @pl.loop(0, dpad, step=16)
      def _(j):
        rbufs[0][i, pl.ds(j, 16)] = jnp.zeros((16,), jnp.float32)

    for r in range(_RPS // _CHUNK):
      pltpu.sync_copy(rbufs[0],
                      acc_sh.at[pl.ds(s * _RPS + r * _CHUNK, _CHUNK)])
    a.wait()
    bb.wait()
    plsc.subcore_barrier()

    pipe_block()

    @pl.when(c == 0)
    def _():
      for g in range(1, _CPW0 // _BPG):
        a, bb = load_idx_start(g)
        a.wait()
        bb.wait()
        pipe_block()

    plsc.subcore_barrier()

    opw = _N // _NS
    pltpu.sync_copy(acc_sh.at[pl.ds(s * opw, opw)],
                    out_hbm.at[c].at[pl.ds(s * opw, opw)])

  return agg


def _lrelu(x):
  return jnp.where(x >= 0, x, 0.01 * x)


def _bn(x):
  m = jnp.mean(x, axis=0, keepdims=True)
  v = jnp.mean((x - m) * (x - m), axis=0, keepdims=True)
  return (x - m) * lax.rsqrt(v + 1e-5)


def _dot(a, b):
  return jnp.dot(a, b, preferred_element_type=jnp.float32)


def _tc_layer1a(aggp, cntp, x, Wl1, bl1, Wr1):
  nb = 5
  rb = _N // nb

  def body(aggp_ref, cntp_ref, x_ref, wl_ref, bl_ref, wr_ref, z_ref, inv_ref):
    cnt = cntp_ref[0][:, 0:1] + cntp_ref[1][:, 0:1]
    inv = 1.0 / jnp.maximum(cnt, 1.0)
    mean = (aggp_ref[0] + aggp_ref[1]) * inv
    z = _dot(mean, wl_ref[...]) + bl_ref[...][None, :] + _dot(x_ref[...], wr_ref[...])
    z_ref[...] = _lrelu(z)
    inv_ref[...] = inv

  return pl.pallas_call(
      body,
      grid=(nb,),
      in_specs=[
          pl.BlockSpec((2, rb, 128), lambda i: (0, i, 0)),
          pl.BlockSpec((2, rb, 16), lambda i: (0, i, 0)),
          pl.BlockSpec((rb, 128), lambda i: (i, 0)),
          pl.BlockSpec((128, 320), lambda i: (0, 0)),
          pl.BlockSpec((320,), lambda i: (0,)),
          pl.BlockSpec((128, 320), lambda i: (0, 0)),
      ],
      out_specs=[
          pl.BlockSpec((rb, 320), lambda i: (i, 0)),
          pl.BlockSpec((rb, 1), lambda i: (i, 0)),
      ],
      out_shape=[
          jax.ShapeDtypeStruct((_N, 320), jnp.float32),
          jax.ShapeDtypeStruct((_N, 1), jnp.float32),
      ],
  )(aggp, cntp, x, Wl1, bl1, Wr1)


def _tc_bn1(z):
  def body(z_ref, y_ref):
    y_ref[...] = _bn(z_ref[...])

  return pl.pallas_call(
      body,
      out_shape=jax.ShapeDtypeStruct((_N, 320), jnp.float32),
  )(z)


def _tc_bn2(aggpa, aggpb, xw, inv, bl):
  def body(aggpa_ref, aggpb_ref, xw_ref, inv_ref, bl_ref, y_ref):
    agg = jnp.concatenate(
        [aggpa_ref[0] + aggpa_ref[1],
         aggpb_ref[0][:, :52] + aggpb_ref[1][:, :52]], axis=1)
    y = agg * inv_ref[...] + bl_ref[...][None, :] + xw_ref[...]
    y_ref[...] = _bn(_lrelu(y))

  return pl.pallas_call(
      body,
      out_shape=jax.ShapeDtypeStruct((_N, 180), jnp.float32),
  )(aggpa, aggpb, xw, inv, bl)


def _tc_bn_mid(aggp, xw, inv, bl, d):
  def body(aggp_ref, xw_ref, inv_ref, bl_ref, y_ref):
    agg = aggp_ref[0][:, :d] + aggp_ref[1][:, :d]
    y = agg * inv_ref[...] + bl_ref[...][None, :] + xw_ref[...]
    y_ref[...] = _bn(_lrelu(y))

  return pl.pallas_call(
      body,
      out_shape=jax.ShapeDtypeStruct((_N, d), jnp.float32),
  )(aggp, xw, inv, bl)


def _tc_proj(y, Wl, Wr, splits, pad_to):
  nb = 5
  rb = _N // nb
  din = y.shape[1]
  dl = Wl.shape[1]
  dr = Wr.shape[1]
  outs = [s[1] - s[0] for s in splits[:-1]] + [pad_to]

  def body(y_ref, wl_ref, wr_ref, *out_refs):
    h = _dot(y_ref[...], wl_ref[...])
    for k, (lo, hi) in enumerate(splits):
      part = h[:, lo:hi]
      if k == len(splits) - 1 and pad_to > hi - lo:
        part = jnp.pad(part, ((0, 0), (0, pad_to - (hi - lo))))
      out_refs[k][...] = part
    out_refs[-1][...] = _dot(y_ref[...], wr_ref[...])

  return pl.pallas_call(
      body,
      grid=(nb,),
      in_specs=[
          pl.BlockSpec((rb, din), lambda i: (i, 0)),
          pl.BlockSpec((din, dl), lambda i: (0, 0)),
          pl.BlockSpec((din, dr), lambda i: (0, 0)),
      ],
      out_specs=[pl.BlockSpec((rb, c), lambda i: (i, 0)) for c in outs]
      + [pl.BlockSpec((rb, dr), lambda i: (i, 0))],
      out_shape=[jax.ShapeDtypeStruct((_N, c), jnp.float32) for c in outs]
      + [jax.ShapeDtypeStruct((_N, dr), jnp.float32)],
  )(y, Wl, Wr)


def _tc_final(y4, fW1, fb1, fW2, fb2, fW3, fb3):
  blen = _N // 16

  def body(y_ref, fw1_ref, fb1_ref, fw2_ref, fb2_ref, fw3_ref, fb3_ref,
           out_ref):
    parts = [jnp.sum(y_ref[pl.ds(b * blen, blen), :], axis=0, keepdims=True)
             for b in range(16)]
    p = jnp.concatenate(parts, axis=0)

    def vdot(a, w):
      return jnp.sum(a[:, :, None] * w[None, :, :], axis=1)

    p = vdot(p, fw1_ref[...]) + fb1_ref[...][None, :]
    p = vdot(p, fw2_ref[...]) + fb2_ref[...][None, :]
    p = vdot(p, fw3_ref[...]) + fb3_ref[...][None, :]
    out_ref[...] = p

  return pl.pallas_call(
      body,
      out_shape=jax.ShapeDtypeStruct((16, 1), jnp.float32),
  )(y4, fW1, fb1, fW2, fb2, fW3, fb3)


_agg16 = _make_sc_aggregate(16, 4)
_agg128 = _make_sc_aggregate(128, 2)
_agg96 = _make_sc_aggregate(96, 4)
_agg64 = _make_sc_aggregate(64, 4)


def kernel(x_in, edge_index, Wl1, bl1, Wr1, Wl2, bl2, Wr2, Wl3, bl3, Wr3,
           Wl4, bl4, Wr4, fW1, fb1, fW2, fb2, fW3, fb3):
  src = jnp.reshape(
      jnp.concatenate([edge_index[0],
                       jnp.zeros((_EP - _E,), jnp.int32)]),
      (_EP // _CHUNK, _CHUNK))
  pad_dst = _N + jnp.arange(_EP - _E, dtype=jnp.int32) % (_NP - _N)
  dst = jnp.reshape(
      jnp.concatenate([edge_index[1], pad_dst]),
      (_EP // _CHUNK, _CHUNK))

  ac = _agg16(jnp.ones((_N, 16), jnp.float32), src, dst)
  a1 = _agg128(x_in, src, dst)
  z1, inv = _tc_layer1a(a1, ac, x_in, Wl1, bl1, Wr1)
  y1 = _tc_bn1(z1)
  h2a, h2b, xw2 = _tc_proj(y1, Wl2, Wr2, [(0, 128), (128, 180)], 64)

  a2a = _agg128(h2a, src, dst)
  a2b = _agg64(h2b, src, dst)
  y2 = _tc_bn2(a2a, a2b, xw2, inv, bl2)
  h3p, xw3 = _tc_proj(y2, Wl3, Wr3, [(0, 90)], 96)

  a3 = _agg96(h3p, src, dst)
  y3 = _tc_bn_mid(a3, xw3, inv, bl3, 90)
  h4p, xw4 = _tc_proj(y3, Wl4, Wr4, [(0, 50)], 64)

  a4 = _agg64(h4p, src, dst)
  y4 = _tc_bn_mid(a4, xw4, inv, bl4, 50)
  return _tc_final(y4, fW1, fb1, fW2, fb2, fW3, fb3)


# speedup vs baseline: 1.1241x; 1.0945x over previous
import functools

import jax
import jax.numpy as jnp
from jax import lax
from jax.experimental import pallas as pl
from jax.experimental.pallas import tpu as pltpu
from jax.experimental.pallas import tpu_sc as plsc

_N = 10000
_NP = 10240
_E = 320000
_NC = 2
_NS = 16
_NW = _NC * _NS
_CHUNK = 128
_CPW0 = 140
_CPW1 = 20
_BPG = 20
_EP = _NS * (_CPW0 + _CPW1) * _CHUNK
_RPS = _NP // _NS


def _make_sc_aggregate(dpad, nbuf):
  mesh = plsc.VectorSubcoreMesh(core_axis_name="c", subcore_axis_name="s")

  @functools.partial(
      pl.kernel,
      mesh=mesh,
      compiler_params=pltpu.CompilerParams(use_tc_tiling_on_sc=False),
      out_type=jax.ShapeDtypeStruct((_NC, _N, dpad), jnp.float32),
      scratch_types=(
          [pltpu.VMEM((_BPG, _CHUNK), jnp.int32),
           pltpu.VMEM((_BPG, _CHUNK), jnp.int32)]
          + [pltpu.VMEM((_CHUNK, dpad), jnp.float32) for _ in range(nbuf)]
          + [pltpu.VMEM_SHARED((_NP, dpad), jnp.float32)]
          + [pltpu.SemaphoreType.DMA for _ in range(2 * nbuf)]
      ),
  )
  def agg(table_hbm, src_hbm, dst_hbm, out_hbm, srcb, dstb, *rest):
    rbufs = rest[:nbuf]
    acc_sh = rest[nbuf]
    sgs = rest[nbuf + 1:2 * nbuf + 1]
    sss = rest[2 * nbuf + 1:]
    c = lax.axis_index("c")
    s = lax.axis_index("s")
    base = jnp.where(c == 0, s * _CPW0, _NS * _CPW0 + s * _CPW1)

    def g_desc(k, b):
      return pltpu.make_async_copy(table_hbm.at[srcb.at[k]], rbufs[b], sgs[b])

    def s_desc(k, b):
      return pltpu.make_async_copy(rbufs[b], acc_sh.at[dstb.at[k]], sss[b])

    def load_idx_start(g):
      a = pltpu.make_async_copy(
          src_hbm.at[pl.ds(base + g * _BPG, _BPG)], srcb, sgs[0])
      bb = pltpu.make_async_copy(
          dst_hbm.at[pl.ds(base + g * _BPG, _BPG)], dstb, sgs[1 % nbuf])
      a.start()
      bb.start()
      return a, bb

    def pipe_block():
      for b in range(nbuf):
        g_desc(b, b).start()

      @pl.loop(0, _BPG // nbuf - 1)
      def _(j):
        k = j * nbuf
        for b in range(nbuf):
          g_desc(k + b, b).wait()
          s_desc(k + b, b).start(add=True)
        for b in range(nbuf):
          s_desc(k + b, b).wait()
          g_desc(k + nbuf + b, b).start()

      tail = _BPG - nbuf
      for b in range(nbuf):
        g_desc(tail + b, b).wait()
        s_desc(tail + b, b).start(add=True)
      for b in range(nbuf):
        s_desc(tail + b, b).wait()

    a, bb = load_idx_start(0)

    @pl.loop(0, _CHUNK)
    def _(i):
      @pl.loop(0, dpad, step=16)
      def _(j):
        rbufs[0][i, pl.ds(j, 16)] = jnp.zeros((16,), jnp.float32)

    for r in range(_RPS // _CHUNK):
      pltpu.sync_copy(rbufs[0],
                      acc_sh.at[pl.ds(s * _RPS + r * _CHUNK, _CHUNK)])
    a.wait()
    bb.wait()
    plsc.subcore_barrier()

    pipe_block()

    @pl.when(c == 0)
    def _():
      for g in range(1, _CPW0 // _BPG):
        a, bb = load_idx_start(g)
        a.wait()
        bb.wait()
        pipe_block()

    plsc.subcore_barrier()

    opw = _N // _NS
    pltpu.sync_copy(acc_sh.at[pl.ds(s * opw, opw)],
                    out_hbm.at[c].at[pl.ds(s * opw, opw)])

  return agg


def _lrelu(x):
  return jnp.where(x >= 0, x, 0.01 * x)


def _bn(x):
  m = jnp.mean(x, axis=0, keepdims=True)
  v = jnp.mean((x - m) * (x - m), axis=0, keepdims=True)
  return (x - m) * lax.rsqrt(v + 1e-5)


def _dot(a, b):
  return jnp.dot(a, b, preferred_element_type=jnp.float32)


def _tc_layer1a(aggp, cntp, x, Wl1, bl1, Wr1):
  nb = 5
  rb = _N // nb

  def body(aggp_ref, cntp_ref, x_ref, wl_ref, bl_ref, wr_ref, z_ref, inv_ref):
    cnt = cntp_ref[0][:, 0:1] + cntp_ref[1][:, 0:1]
    inv = 1.0 / jnp.maximum(cnt, 1.0)
    mean = (aggp_ref[0] + aggp_ref[1]) * inv
    z = _dot(mean, wl_ref[...]) + bl_ref[...][None, :] + _dot(x_ref[...], wr_ref[...])
    z_ref[...] = _lrelu(z)
    inv_ref[...] = inv

  return pl.pallas_call(
      body,
      grid=(nb,),
      in_specs=[
          pl.BlockSpec((2, rb, 128), lambda i: (0, i, 0)),
          pl.BlockSpec((2, rb, 16), lambda i: (0, i, 0)),
          pl.BlockSpec((rb, 128), lambda i: (i, 0)),
          pl.BlockSpec((128, 320), lambda i: (0, 0)),
          pl.BlockSpec((320,), lambda i: (0,)),
          pl.BlockSpec((128, 320), lambda i: (0, 0)),
      ],
      out_specs=[
          pl.BlockSpec((rb, 320), lambda i: (i, 0)),
          pl.BlockSpec((rb, 1), lambda i: (i, 0)),
      ],
      out_shape=[
          jax.ShapeDtypeStruct((_N, 320), jnp.float32),
          jax.ShapeDtypeStruct((_N, 1), jnp.float32),
      ],
  )(aggp, cntp, x, Wl1, bl1, Wr1)


def _tc_bn1(z):
  def body(z_ref, y_ref):
    y_ref[...] = _bn(z_ref[...])

  return pl.pallas_call(
      body,
      out_shape=jax.ShapeDtypeStruct((_N, 320), jnp.float32),
  )(z)


def _tc_bn2(aggpa, aggpb, xw, inv, bl):
  def body(aggpa_ref, aggpb_ref, xw_ref, inv_ref, bl_ref, y_ref):
    agg = jnp.concatenate(
        [aggpa_ref[0] + aggpa_ref[1],
         aggpb_ref[0][:, :52] + aggpb_ref[1][:, :52]], axis=1)
    y = agg * inv_ref[...] + bl_ref[...][None, :] + xw_ref[...]
    y_ref[...] = _bn(_lrelu(y))

  return pl.pallas_call(
      body,
      out_shape=jax.ShapeDtypeStruct((_N, 180), jnp.float32),
  )(aggpa, aggpb, xw, inv, bl)


def _tc_bn_mid(aggp, xw, inv, bl, d):
  def body(aggp_ref, xw_ref, inv_ref, bl_ref, y_ref):
    agg = aggp_ref[0][:, :d] + aggp_ref[1][:, :d]
    y = agg * inv_ref[...] + bl_ref[...][None, :] + xw_ref[...]
    y_ref[...] = _bn(_lrelu(y))

  return pl.pallas_call(
      body,
      out_shape=jax.ShapeDtypeStruct((_N, d), jnp.float32),
  )(aggp, xw, inv, bl)


def _tc_proj(y, Wl, Wr, splits, pad_to):
  nb = 5
  rb = _N // nb
  din = y.shape[1]
  dl = Wl.shape[1]
  dr = Wr.shape[1]
  outs = [s[1] - s[0] for s in splits[:-1]] + [pad_to]

  def body(y_ref, wl_ref, wr_ref, *out_refs):
    h = _dot(y_ref[...], wl_ref[...])
    for k, (lo, hi) in enumerate(splits):
      part = h[:, lo:hi]
      if k == len(splits) - 1 and pad_to > hi - lo:
        part = jnp.pad(part, ((0, 0), (0, pad_to - (hi - lo))))
      out_refs[k][...] = part
    out_refs[-1][...] = _dot(y_ref[...], wr_ref[...])

  return pl.pallas_call(
      body,
      grid=(nb,),
      in_specs=[
          pl.BlockSpec((rb, din), lambda i: (i, 0)),
          pl.BlockSpec((din, dl), lambda i: (0, 0)),
          pl.BlockSpec((din, dr), lambda i: (0, 0)),
      ],
      out_specs=[pl.BlockSpec((rb, c), lambda i: (i, 0)) for c in outs]
      + [pl.BlockSpec((rb, dr), lambda i: (i, 0))],
      out_shape=[jax.ShapeDtypeStruct((_N, c), jnp.float32) for c in outs]
      + [jax.ShapeDtypeStruct((_N, dr), jnp.float32)],
  )(y, Wl, Wr)


def _tc_final(y4, fW1, fb1, fW2, fb2, fW3, fb3):
  blen = _N // 16

  def body(y_ref, fw1_ref, fb1_ref, fw2_ref, fb2_ref, fw3_ref, fb3_ref,
           out_ref):
    parts = [jnp.sum(y_ref[pl.ds(b * blen, blen), :], axis=0, keepdims=True)
             for b in range(16)]
    p = jnp.concatenate(parts, axis=0)

    def vdot(a, w):
      return jnp.sum(a[:, :, None] * w[None, :, :], axis=1)

    p = vdot(p, fw1_ref[...]) + fb1_ref[...][None, :]
    p = vdot(p, fw2_ref[...]) + fb2_ref[...][None, :]
    p = vdot(p, fw3_ref[...]) + fb3_ref[...][None, :]
    out_ref[...] = p

  return pl.pallas_call(
      body,
      out_shape=jax.ShapeDtypeStruct((16, 1), jnp.float32),
  )(y4, fW1, fb1, fW2, fb2, fW3, fb3)


_agg16 = _make_sc_aggregate(16, 4)
_agg128 = _make_sc_aggregate(128, 2)
_agg96 = _make_sc_aggregate(96, 4)
_agg64 = _make_sc_aggregate(64, 4)


def kernel(x_in, edge_index, Wl1, bl1, Wr1, Wl2, bl2, Wr2, Wl3, bl3, Wr3,
           Wl4, bl4, Wr4, fW1, fb1, fW2, fb2, fW3, fb3):
  src = jnp.reshape(
      jnp.concatenate([edge_index[0],
                       jnp.zeros((_EP - _E,), jnp.int32)]),
      (_EP // _CHUNK, _CHUNK))
  pad_dst = _N + jnp.arange(_EP - _E, dtype=jnp.int32) % (_NP - _N)
  dst = jnp.reshape(
      jnp.concatenate([edge_index[1], pad_dst]),
      (_EP // _CHUNK, _CHUNK))

  ac = _agg16(jnp.ones((_N, 16), jnp.float32), src, dst)
  a1 = _agg128(x_in, src, dst)
  z1, inv = _tc_layer1a(a1, ac, x_in, Wl1, bl1, Wr1)
  y1 = _tc_bn1(z1)
  h2a, h2b, xw2 = _tc_proj(y1, Wl2, Wr2, [(0, 128), (128, 180)], 64)

  a2a = _agg128(h2a, src, dst)
  a2b = _agg64(h2b, src, dst)
  y2 = _tc_bn2(a2a, a2b, xw2, inv, bl2)
  h3p, xw3 = _tc_proj(y2, Wl3, Wr3, [(0, 90)], 96)

  a3 = _agg96(h3p, src, dst)
  y3 = _tc_bn_mid(a3, xw3, inv, bl3, 90)
  h4p, xw4 = _tc_proj(y3, Wl4, Wr4, [(0, 50)], 64)

  a4 = _agg64(h4p, src, dst)
  y4 = _tc_bn_mid(a4, xw4, inv, bl4, 50)
  return _tc_final(y4, fW1, fb1, fW2, fb2, fW3, fb3)
